# baseline (speedup 1.0000x reference)
import functools
import math

import jax
import jax.numpy as jnp
from jax import lax
from jax.experimental import pallas as pl
from jax.experimental.pallas import tpu as pltpu
from jax.experimental.pallas import tpu_sc as plsc

D_MODEL = 64
B = 4096
L = 200
NW = 32
BW = B // NW
NBUF = 5
NWB = 5
SCALE = math.sqrt(float(D_MODEL))
OUT_ROWS = L * NW * D_MODEL

_mesh = plsc.VectorSubcoreMesh(core_axis_name="c", subcore_axis_name="s")


@functools.partial(
    pl.kernel,
    mesh=_mesh,
    compiler_params=pltpu.CompilerParams(
        needs_layout_passes=False, use_tc_tiling_on_sc=False
    ),
    out_type=jax.ShapeDtypeStruct((OUT_ROWS, 128), jnp.float32),
    scratch_types=[
        pltpu.VMEM((L, BW), jnp.int32),
        pltpu.VMEM((BW,), jnp.int32),
        pltpu.VMEM((16, D_MODEL), jnp.float32),
        pltpu.VMEM((NBUF, BW, D_MODEL), jnp.float32),
        pltpu.VMEM((NWB, D_MODEL, BW), jnp.float32),
        pltpu.SemaphoreType.DMA((NBUF,)),
        pltpu.SemaphoreType.DMA((NWB,)),
    ],
)
def _embed_sc(task0_hbm, uniT_hbm, ttab_hbm, utab_hbm, out_hbm,
              uidx_v, tidx_v, ttab_v, gbuf_v, obuf_v, gsem, wsem):
    wid = lax.axis_index("s") * 2 + lax.axis_index("c")
    b0 = wid * BW

    pltpu.sync_copy(uniT_hbm.at[:, pl.ds(b0, BW)], uidx_v)
    pltpu.sync_copy(task0_hbm.at[pl.ds(b0, BW)], tidx_v)
    pltpu.sync_copy(ttab_hbm, ttab_v)

    def dbl(t, c):
        for m in range(BW // 16):
            sl = pl.ds(16 * m, 16)
            uidx_v[t, sl] = uidx_v[t, sl] * 2
        return c

    lax.fori_loop(0, L, dbl, 0)

    for r in range(16):
        for m in range(D_MODEL // 16):
            sl = pl.ds(16 * m, 16)
            ttab_v[r, sl] = ttab_v[r, sl] * SCALE

    lanes = lax.iota(jnp.int32, 16)
    rows_m = [lanes + 16 * m for m in range(BW // 16)]

    def gather(t, p):
        pltpu.make_async_copy(
            utab_hbm.at[uidx_v.at[t]], gbuf_v.at[p], gsem.at[p]
        ).start()

    def gather_wait(t, p):
        pltpu.make_async_copy(
            utab_hbm.at[uidx_v.at[t]], gbuf_v.at[p], gsem.at[p]
        ).wait()

    def write(t, p):
        base = (t * NW + wid) * D_MODEL
        pltpu.make_async_copy(
            obuf_v.at[p], out_hbm.at[pl.ds(base, D_MODEL)], wsem.at[p]
        ).start()

    def write_wait(t, p):
        base = (t * NW + wid) * D_MODEL
        pltpu.make_async_copy(
            obuf_v.at[p], out_hbm.at[pl.ds(base, D_MODEL)], wsem.at[p]
        ).wait()

    def transpose_block(p, src, scale):
        obuf2d = obuf_v.at[p]

        def mbody(m, c):
            rows = lanes + m * 16
            for k0 in range(D_MODEL):
                kcol = jnp.bitwise_and(lanes + k0, D_MODEL - 1)
                v = plsc.load_gather(src, [rows, kcol])
                if scale:
                    v = v * SCALE
                plsc.store_scatter(obuf2d, [kcol, rows], v)
            return c

        lax.fori_loop(0, BW // 16, mbody, 0)

    for p in range(NBUF):
        gather(p, p)

    def step(j, carry):
        for p in range(NBUF):
            t = NBUF * j + p
            q = p % NWB
            gather_wait(t, p)

            @pl.when(t >= NWB)
            def _():
                write_wait(t - NWB, q)

            transpose_block(q, gbuf_v.at[p], True)
            write(t, q)

            @pl.when(j < L // NBUF - 1)
            def _():
                gather(t + NBUF, p)
        return carry

    lax.fori_loop(0, L // NBUF, step, 0)
    for q in range(NWB):
        write_wait(L - NWB + q, q)

    obuf0 = obuf_v.at[0]

    def t0body(m, c):
        rows = lanes + m * 16
        tvec = tidx_v[pl.ds(m * 16, 16)]
        for k0 in range(D_MODEL):
            kcol = jnp.bitwise_and(lanes + k0, D_MODEL - 1)
            v = plsc.load_gather(ttab_v, [tvec, kcol])
            plsc.store_scatter(obuf0, [kcol, rows], v)
        return c

    lax.fori_loop(0, BW // 16, t0body, 0)
    write(0, 0)
    write_wait(0, 0)


def kernel(task, uni, task_table, uni_table):
    task0 = task[:, 0]
    uniT = uni.T
    utab2 = jnp.pad(uni_table, ((0, 0), (0, 64))).reshape(2 * 1000000, D_MODEL)
    out = _embed_sc(task0, uniT, task_table, utab2)
    x = out.reshape(L, NW, D_MODEL, 128)
    return x.transpose(1, 3, 0, 2).reshape(B, L, D_MODEL)


# speedup vs baseline: 1.1693x; 1.1693x over previous
import functools
import math

import jax
import jax.numpy as jnp
from jax import lax
from jax.experimental import pallas as pl
from jax.experimental.pallas import tpu as pltpu
from jax.experimental.pallas import tpu_sc as plsc

D_MODEL = 64
B = 4096
L = 200
NW = 32
BW = B // NW
NBUF = 8
AHEAD = 4
SCALE = math.sqrt(float(D_MODEL))

_mesh = plsc.VectorSubcoreMesh(core_axis_name="c", subcore_axis_name="s")


@functools.partial(
    pl.kernel,
    mesh=_mesh,
    compiler_params=pltpu.CompilerParams(
        needs_layout_passes=False, use_tc_tiling_on_sc=False
    ),
    out_type=jax.ShapeDtypeStruct((L * B, D_MODEL), jnp.float32),
    scratch_types=[
        pltpu.VMEM((L, BW), jnp.int32),
        pltpu.VMEM((BW,), jnp.int32),
        pltpu.VMEM((16, D_MODEL), jnp.float32),
        pltpu.VMEM((NBUF, BW, D_MODEL), jnp.float32),
        pltpu.SemaphoreType.DMA((NBUF,)),
        pltpu.SemaphoreType.DMA((NBUF,)),
    ],
)
def _embed_sc(task0_hbm, uniT_hbm, ttab_hbm, utab_hbm, out_hbm,
              uidx_v, tidx_v, ttab_v, gbuf_v, gsem, wsem):
    wid = lax.axis_index("s") * 2 + lax.axis_index("c")
    b0 = wid * BW

    pltpu.sync_copy(uniT_hbm.at[:, pl.ds(b0, BW)], uidx_v)
    pltpu.sync_copy(task0_hbm.at[pl.ds(b0, BW)], tidx_v)
    pltpu.sync_copy(ttab_hbm, ttab_v)

    def dbl(t, c):
        for m in range(BW // 16):
            sl = pl.ds(16 * m, 16)
            uidx_v[t, sl] = uidx_v[t, sl] * 2
        return c

    lax.fori_loop(0, L, dbl, 0)

    for r in range(16):
        for m in range(D_MODEL // 16):
            sl = pl.ds(16 * m, 16)
            ttab_v[r, sl] = ttab_v[r, sl] * SCALE

    lanes = lax.iota(jnp.int32, 16)

    def gather(t, p):
        pltpu.make_async_copy(
            utab_hbm.at[uidx_v.at[t]], gbuf_v.at[p], gsem.at[p]
        ).start()

    def gather_wait(t, p):
        pltpu.make_async_copy(
            utab_hbm.at[uidx_v.at[t]], gbuf_v.at[p], gsem.at[p]
        ).wait()

    def write(t, p):
        pltpu.make_async_copy(
            gbuf_v.at[p], out_hbm.at[pl.ds(t * B + b0, BW)], wsem.at[p]
        ).start()

    def write_wait(t, p):
        pltpu.make_async_copy(
            gbuf_v.at[p], out_hbm.at[pl.ds(t * B + b0, BW)], wsem.at[p]
        ).wait()

    def scale_block(p):
        def sbody(r4, c):
            for dr in range(4):
                for m in range(D_MODEL // 16):
                    sl = pl.ds(16 * m, 16)
                    gbuf_v[p, r4 * 4 + dr, sl] = gbuf_v[p, r4 * 4 + dr, sl] * SCALE
            return c

        lax.fori_loop(0, BW // 4, sbody, 0)

    for p in range(AHEAD):
        gather(p, p)

    def step(j, carry):
        for p in range(NBUF):
            t = NBUF * j + p
            gather_wait(t, p)
            scale_block(p)
            write(t, p)
            tn = t + AHEAD
            pn = (p + AHEAD) % NBUF

            @pl.when(tn >= NBUF)
            def _():
                write_wait(tn - NBUF, pn)

            @pl.when(tn < L)
            def _():
                gather(tn, pn)
        return carry

    lax.fori_loop(0, L // NBUF, step, 0)
    for p in range(NBUF - AHEAD, NBUF):
        write_wait(L - NBUF + p, p)

    gbuf0 = gbuf_v.at[0]

    def t0body(m, c):
        rows = lanes + m * 16
        tvec = tidx_v[pl.ds(m * 16, 16)]
        for k0 in range(D_MODEL):
            kcol = jnp.bitwise_and(lanes + k0, D_MODEL - 1)
            v = plsc.load_gather(ttab_v, [tvec, kcol])
            plsc.store_scatter(gbuf0, [rows, kcol], v)
        return c

    lax.fori_loop(0, BW // 16, t0body, 0)
    write(0, 0)
    write_wait(0, 0)


def kernel(task, uni, task_table, uni_table):
    task0 = task[:, 0]
    uniT = uni.T
    utab2 = jnp.pad(uni_table, ((0, 0), (0, 64))).reshape(2 * 1000000, D_MODEL)
    out = _embed_sc(task0, uniT, task_table, utab2)
    return out.reshape(L, B, D_MODEL).transpose(1, 0, 2)


# speedup vs baseline: 1.5974x; 1.3661x over previous
import functools
import math

import jax
import jax.numpy as jnp
from jax import lax
from jax.experimental import pallas as pl
from jax.experimental.pallas import tpu as pltpu
from jax.experimental.pallas import tpu_sc as plsc

D_MODEL = 64
B = 4096
L = 200
NW = 32
BW = B // NW
NBUF = 8
AHEAD = 4
SCALE = math.sqrt(float(D_MODEL))

_mesh = plsc.VectorSubcoreMesh(core_axis_name="c", subcore_axis_name="s")


@functools.partial(
    pl.kernel,
    mesh=_mesh,
    compiler_params=pltpu.CompilerParams(
        needs_layout_passes=False, use_tc_tiling_on_sc=False
    ),
    out_type=jax.ShapeDtypeStruct((L * B, 128), jnp.float32),
    scratch_types=[
        pltpu.VMEM((L, BW), jnp.int32),
        pltpu.VMEM((BW,), jnp.int32),
        pltpu.VMEM((16, D_MODEL), jnp.float32),
        pltpu.VMEM((NBUF, BW, D_MODEL), jnp.float32),
        pltpu.SemaphoreType.DMA((NBUF,)),
        pltpu.SemaphoreType.DMA((NBUF,)),
    ],
)
def _embed_sc(task0_hbm, uniT_hbm, ttab_hbm, utab_hbm, out_hbm,
              uidx_v, tidx_v, ttab_v, gbuf_v, gsem, wsem):
    wid = lax.axis_index("s") * 2 + lax.axis_index("c")
    b0 = wid * BW

    pltpu.sync_copy(uniT_hbm.at[:, pl.ds(b0, BW)], uidx_v)
    pltpu.sync_copy(task0_hbm.at[pl.ds(b0, BW)], tidx_v)
    pltpu.sync_copy(ttab_hbm, ttab_v)

    def dbl(t, c):
        for m in range(BW // 16):
            sl = pl.ds(16 * m, 16)
            uidx_v[t, sl] = uidx_v[t, sl] * 2
        return c

    lax.fori_loop(0, L, dbl, 0)

    for r in range(16):
        for m in range(D_MODEL // 16):
            sl = pl.ds(16 * m, 16)
            ttab_v[r, sl] = ttab_v[r, sl] * SCALE

    lanes = lax.iota(jnp.int32, 16)

    def gather(t, p):
        pltpu.make_async_copy(
            utab_hbm.at[uidx_v.at[t]], gbuf_v.at[p], gsem.at[p]
        ).start()

    def gather_wait(t, p):
        pltpu.make_async_copy(
            utab_hbm.at[uidx_v.at[t]], gbuf_v.at[p], gsem.at[p]
        ).wait()

    def write(t, p):
        pltpu.make_async_copy(
            gbuf_v.at[p],
            out_hbm.at[pl.ds(t * B + b0, BW), pl.ds(0, D_MODEL)],
            wsem.at[p],
        ).start()

    def write_wait(t, p):
        pltpu.make_async_copy(
            gbuf_v.at[p],
            out_hbm.at[pl.ds(t * B + b0, BW), pl.ds(0, D_MODEL)],
            wsem.at[p],
        ).wait()

    def scale_block(p):
        def sbody(r4, c):
            for dr in range(4):
                for m in range(D_MODEL // 16):
                    sl = pl.ds(16 * m, 16)
                    gbuf_v[p, r4 * 4 + dr, sl] = gbuf_v[p, r4 * 4 + dr, sl] * SCALE
            return c

        lax.fori_loop(0, BW // 4, sbody, 0)

    for p in range(AHEAD):
        gather(p, p)

    def step(j, carry):
        for p in range(NBUF):
            t = NBUF * j + p
            gather_wait(t, p)
            scale_block(p)
            write(t, p)
            tn = t + AHEAD
            pn = (p + AHEAD) % NBUF

            @pl.when(tn >= NBUF)
            def _():
                write_wait(tn - NBUF, pn)

            @pl.when(tn < L)
            def _():
                gather(tn, pn)
        return carry

    lax.fori_loop(0, L // NBUF, step, 0)
    for p in range(NBUF - AHEAD, NBUF):
        write_wait(L - NBUF + p, p)

    gbuf0 = gbuf_v.at[0]

    def t0body(m, c):
        rows = lanes + m * 16
        tvec = tidx_v[pl.ds(m * 16, 16)]
        for k0 in range(D_MODEL):
            kcol = jnp.bitwise_and(lanes + k0, D_MODEL - 1)
            v = plsc.load_gather(ttab_v, [tvec, kcol])
            plsc.store_scatter(gbuf0, [rows, kcol], v)
        return c

    lax.fori_loop(0, BW // 16, t0body, 0)
    write(0, 0)
    write_wait(0, 0)


def kernel(task, uni, task_table, uni_table):
    task0 = task[:, 0]
    uniT = uni.T
    utab2 = jnp.pad(uni_table, ((0, 0), (0, 64))).reshape(2 * 1000000, D_MODEL)
    out = _embed_sc(task0, uniT, task_table, utab2)
    return out[:, :D_MODEL].reshape(L, B, D_MODEL).transpose(1, 0, 2)
